# baseline (speedup 1.0000x reference)
import functools

import jax
import jax.numpy as jnp
from jax import lax
from jax.experimental import pallas as pl
from jax.experimental.pallas import tpu as pltpu
from jax.experimental.pallas import tpu_sc as plsc

_TAU = 0.07
_N_STATES = 4096


def _leaky_relu(x):
    return jnp.where(x >= 0, x, 0.01 * x)


def _cosine_sim(a, b, eps=1e-8):
    num = jnp.sum(a * b, axis=1)
    na = jnp.maximum(jnp.sqrt(jnp.sum(a * a, axis=1)), eps)
    nb = jnp.maximum(jnp.sqrt(jnp.sum(b * b, axis=1)), eps)
    return num / (na * nb)


def _row_col_to_seq(rc, num_cols):
    return (rc[:, 0] * num_cols + rc[:, 1]).astype(jnp.int32)


def _sample_middle_state(base_key, s_, g_, code, ms_w1, ms_b1, ms_w2, ms_b2,
                         states, num_cols):
    k_noise = jax.random.fold_in(base_key, 2 * code)
    k_gumbel = jax.random.fold_in(base_key, 2 * code + 1)
    noise = jax.random.normal(k_noise, s_.shape, dtype=jnp.float32)
    x = jnp.concatenate([s_, g_, noise], axis=1)
    m = _leaky_relu(x @ ms_w1 + ms_b1) @ ms_w2 + ms_b2
    rep = jnp.tile(m, (states.shape[0], 1))
    sim = _cosine_sim(rep, states)
    u = jax.random.uniform(k_gumbel, sim.shape, minval=1e-10, maxval=1.0)
    gn = -jnp.log(-jnp.log(u))
    z = (sim + gn) / _TAU
    zmax = jnp.max(z, -1, initial=-jnp.inf, keepdims=True)
    unnorm = jnp.exp(z - jax.lax.stop_gradient(zmax))
    ssum = jnp.sum(unnorm, -1, keepdims=True)[0]
    am = jnp.argmax(unnorm)
    ya = 1.0 / ssum
    pa = (1.0 + ya) - ya
    m2 = pa * states[am]
    m_idx = _row_col_to_seq(m2[None, :], num_cols)
    return m2[None, :], m_idx


_NC, _NS = 2, 16


def _sc_kernel(sf_hbm, si_hbm, w1_hbm, b1_hbm, w2_hbm, vf_hbm, out_hbm,
               sf, si, w1, b1, w2, idxv, vg, ob, sem):
    cid = lax.axis_index("c")
    sid = lax.axis_index("s")

    @pl.when(jnp.logical_and(cid == 0, sid == 0))
    def _():
        pltpu.sync_copy(sf_hbm, sf)
        pltpu.sync_copy(si_hbm, si)
        pltpu.sync_copy(w1_hbm, w1)
        pltpu.sync_copy(b1_hbm, b1)
        pltpu.sync_copy(w2_hbm, w2)

        sfv = sf[...]
        s0 = sfv[0]
        s1 = sfv[1]
        g0 = sfv[2]
        g1 = sfv[3]
        rA = sfv[4]
        cA = sfv[5]
        rB = sfv[6]
        cB = sfv[7]
        rC = sfv[8]
        cC = sfv[9]
        eb2 = sfv[10]

        siv = si[...]
        sidx = siv[0]
        gidx = siv[1]
        iA = siv[2]
        iB = siv[3]
        iC = siv[4]

        pairs = [
            (sidx, gidx),
            (sidx, iA),
            (sidx, iB),
            (iB, iA),
            (iA, gidx),
            (iA, iC),
            (iC, gidx),
        ]
        io = lax.iota(jnp.int32, 16)
        rows = jnp.zeros((16,), jnp.int32)
        for k, (a, _) in enumerate(pairs):
            rows = jnp.where(io == k, a, rows)
        idxv[...] = rows
        cp = pltpu.make_async_copy(vf_hbm.at[idxv], vg, sem)
        cp.start()

        xs = [
            (s0, s1, g0, g1),
            (s0, s1, rA, cA),
            (s0, s1, rB, cB),
            (rB, cB, rA, cA),
            (rA, cA, g0, g1),
            (rA, cA, rC, cC),
            (rC, cC, g0, g1),
        ]
        def _round_bf16(v):
            c = jnp.float32(65537.0)
            p = v * c
            q = p - v
            return p - q

        accs = [jnp.zeros((16,), jnp.float32) for _ in range(7)]
        for j in range(16):
            sl = pl.ds(j * 16, 16)
            w1r = [w1[k, sl] for k in range(4)]
            bj = b1[sl]
            w2j = w2[sl]
            for t in range(7):
                x0, x1, x2, x3 = xs[t]
                h = bj + x0 * w1r[0] + x1 * w1r[1] + x2 * w1r[2] + x3 * w1r[3]
                h = jnp.where(h >= 0, h, jnp.float32(0.01) * h)
                accs[t] = accs[t] + _round_bf16(h) * w2j
        def _lanesum(a):
            parts = [a[k] for k in range(16)]
            while len(parts) > 1:
                parts = [parts[i] + parts[i + 1]
                         for i in range(0, len(parts), 2)]
            return parts[0]

        ys = [_lanesum(a) + eb2 for a in accs]

        cp.wait()
        vs = []
        for k, (_, b) in enumerate(pairs):
            base = pl.multiple_of((b >> 4) << 4, 16)
            lane = b & 15
            vec = vg[k, pl.ds(base, 16)]
            parts = [vec[j] for j in range(16)]
            val = parts[0]
            for j in range(1, 16):
                val = jnp.where(lane == j, parts[j], val)
            vs.append(val)
        ls = [jnp.abs(y - v) for y, v in zip(ys, vs)]

        condA = jnp.logical_or(iA == gidx, iA == sidx)
        condB = jnp.logical_or(iB == iA, iB == sidx)
        condC = jnp.logical_or(iC == gidx, iC == iA)

        def comb(vals):
            resB = jnp.where(condB, vals[1], vals[2] + vals[3])
            resC = jnp.where(condC, vals[4], vals[5] + vals[6])
            return jnp.where(condA, vals[0], resB + resC)

        yo = comb(ys)
        vo = comb(vs)
        lo = comb(ls)
        out = jnp.where(io == 0, yo, jnp.where(io == 1, vo,
                        jnp.where(io == 2, lo, jnp.float32(0.0))))
        ob[...] = out
        pltpu.sync_copy(ob, out_hbm)


_sc_call = functools.partial(
    pl.kernel,
    out_type=jax.ShapeDtypeStruct((16,), jnp.float32),
    mesh=plsc.VectorSubcoreMesh(core_axis_name="c", subcore_axis_name="s",
                                num_cores=_NC, num_subcores=_NS),
    scratch_types=[
        pltpu.VMEM((16,), jnp.float32),
        pltpu.VMEM((16,), jnp.int32),
        pltpu.VMEM((4, 256), jnp.float32),
        pltpu.VMEM((256,), jnp.float32),
        pltpu.VMEM((256,), jnp.float32),
        pltpu.VMEM((16,), jnp.int32),
        pltpu.VMEM((16, 4096), jnp.float32),
        pltpu.VMEM((16,), jnp.float32),
        pltpu.SemaphoreType.DMA,
    ],
)(_sc_kernel)


def kernel(s, g, s_idx, g_idx, ev_w1, ev_b1, ev_w2, ev_b2, ms_w1, ms_b1,
           ms_w2, ms_b2, states, value_function, num_cols):
    base_key = jax.random.key(42)

    mA, iA = _sample_middle_state(base_key, s, g, 0, ms_w1, ms_b1, ms_w2,
                                  ms_b2, states, num_cols)
    mB, iB = _sample_middle_state(base_key, s, mA, 1, ms_w1, ms_b1, ms_w2,
                                  ms_b2, states, num_cols)
    mC, iC = _sample_middle_state(base_key, mA, g, 2, ms_w1, ms_b1, ms_w2,
                                  ms_b2, states, num_cols)

    def _bf(v):
        return v.astype(jnp.bfloat16).astype(jnp.float32)

    sf = jnp.concatenate([
        _bf(s.reshape(-1)), _bf(g.reshape(-1)), _bf(mA.reshape(-1)),
        _bf(mB.reshape(-1)), _bf(mC.reshape(-1)), ev_b2.reshape(-1),
        jnp.zeros((5,), jnp.float32),
    ]).astype(jnp.float32)
    si = jnp.concatenate([
        s_idx.reshape(-1), g_idx.reshape(-1), iA.reshape(-1), iB.reshape(-1),
        iC.reshape(-1), jnp.zeros((11,), jnp.int32),
    ]).astype(jnp.int32)

    out = _sc_call(sf, si, _bf(ev_w1), ev_b1, _bf(ev_w2.reshape(-1)),
                   value_function)

    y = out[0:1].reshape(1, 1)
    v = out[1:2]
    loss = out[2]
    return y, v, loss


# speedup vs baseline: 1.0576x; 1.0576x over previous
import functools

import jax
import jax.numpy as jnp
from jax import lax
from jax.experimental import pallas as pl
from jax.experimental.pallas import tpu as pltpu
from jax.experimental.pallas import tpu_sc as plsc

_TAU = 0.07
_N_STATES = 4096


def _leaky_relu(x):
    return jnp.where(x >= 0, x, 0.01 * x)


def _cosine_sim(a, b, eps=1e-8):
    num = jnp.sum(a * b, axis=1)
    na = jnp.maximum(jnp.sqrt(jnp.sum(a * a, axis=1)), eps)
    nb = jnp.maximum(jnp.sqrt(jnp.sum(b * b, axis=1)), eps)
    return num / (na * nb)


def _row_col_to_seq(rc, num_cols):
    return (rc[:, 0] * num_cols + rc[:, 1]).astype(jnp.int32)


def _sample_middle_state(base_key, s_, g_, code, ms_w1, ms_b1, ms_w2, ms_b2,
                         states, num_cols):
    k_noise = jax.random.fold_in(base_key, 2 * code)
    k_gumbel = jax.random.fold_in(base_key, 2 * code + 1)
    noise = jax.random.normal(k_noise, s_.shape, dtype=jnp.float32)
    x = jnp.concatenate([s_, g_, noise], axis=1)
    m = _leaky_relu(x @ ms_w1 + ms_b1) @ ms_w2 + ms_b2
    rep = jnp.tile(m, (states.shape[0], 1))
    sim = _cosine_sim(rep, states)
    u = jax.random.uniform(k_gumbel, sim.shape, minval=1e-10, maxval=1.0)
    gn = -jnp.log(-jnp.log(u))
    z = (sim + gn) / _TAU
    zmax = jnp.max(z, -1, initial=-jnp.inf, keepdims=True)
    unnorm = jnp.exp(z - jax.lax.stop_gradient(zmax))
    ssum = jnp.sum(unnorm, -1, keepdims=True)[0]
    am = jnp.argmax(unnorm)
    ya = 1.0 / ssum
    pa = (1.0 + ya) - ya
    m2 = pa * states[am]
    m_idx = _row_col_to_seq(m2[None, :], num_cols)
    return m2[None, :], m_idx


_NC, _NS = 2, 16


def _sc_kernel(sf_hbm, si_hbm, wp_hbm, vf_hbm, out_hbm,
               sf, si, wp, idxv, vg, ob, semw, semg):
    cid = lax.axis_index("c")
    sid = lax.axis_index("s")

    @pl.when(jnp.logical_and(cid == 0, sid == 0))
    def _():
        cpw = pltpu.make_async_copy(wp_hbm, wp, semw)
        cpw.start()
        pltpu.sync_copy(sf_hbm, sf)
        pltpu.sync_copy(si_hbm, si)

        sfv = sf[...]
        s0 = sfv[0]
        s1 = sfv[1]
        g0 = sfv[2]
        g1 = sfv[3]
        rA = sfv[4]
        cA = sfv[5]
        rB = sfv[6]
        cB = sfv[7]
        rC = sfv[8]
        cC = sfv[9]
        eb2 = sfv[10]

        siv = si[...]
        sidx = siv[0]
        gidx = siv[1]
        iA = siv[2]
        iB = siv[3]
        iC = siv[4]

        pairs = [
            (sidx, gidx),
            (sidx, iA),
            (sidx, iB),
            (iB, iA),
            (iA, gidx),
            (iA, iC),
            (iC, gidx),
        ]
        io = lax.iota(jnp.int32, 16)
        rows = jnp.zeros((16,), jnp.int32)
        for k, (a, b) in enumerate(pairs):
            rows = jnp.where(io == k, a * jnp.int32(_N_STATES) + b, rows)
        idxv[...] = rows
        cpg = pltpu.make_async_copy(vf_hbm.at[idxv], vg, semg)
        cpg.start()

        xs = [
            (s0, s1, g0, g1),
            (s0, s1, rA, cA),
            (s0, s1, rB, cB),
            (rB, cB, rA, cA),
            (rA, cA, g0, g1),
            (rA, cA, rC, cC),
            (rC, cC, g0, g1),
        ]
        def _round_bf16(v):
            c = jnp.float32(65537.0)
            p = v * c
            q = p - v
            return p - q

        cpw.wait()
        accs = [jnp.zeros((16,), jnp.float32) for _ in range(7)]
        for j in range(16):
            sl = pl.ds(j * 16, 16)
            w1r = [wp[k, sl] for k in range(4)]
            bj = wp[4, sl]
            w2j = wp[5, sl]
            for t in range(7):
                x0, x1, x2, x3 = xs[t]
                h = bj + x0 * w1r[0] + x1 * w1r[1] + x2 * w1r[2] + x3 * w1r[3]
                h = jnp.where(h >= 0, h, jnp.float32(0.01) * h)
                accs[t] = accs[t] + _round_bf16(h) * w2j
        def _lanesum(a):
            parts = [a[k] for k in range(16)]
            while len(parts) > 1:
                parts = [parts[i] + parts[i + 1]
                         for i in range(0, len(parts), 2)]
            return parts[0]

        ys = [_lanesum(a) + eb2 for a in accs]

        cpg.wait()
        vgv = vg[...]
        vs = [vgv[k] for k in range(7)]
        ls = [jnp.abs(y - v) for y, v in zip(ys, vs)]

        condA = jnp.logical_or(iA == gidx, iA == sidx)
        condB = jnp.logical_or(iB == iA, iB == sidx)
        condC = jnp.logical_or(iC == gidx, iC == iA)

        def comb(vals):
            resB = jnp.where(condB, vals[1], vals[2] + vals[3])
            resC = jnp.where(condC, vals[4], vals[5] + vals[6])
            return jnp.where(condA, vals[0], resB + resC)

        yo = comb(ys)
        vo = comb(vs)
        lo = comb(ls)
        out = jnp.where(io == 0, yo, jnp.where(io == 1, vo,
                        jnp.where(io == 2, lo, jnp.float32(0.0))))
        ob[...] = out
        pltpu.sync_copy(ob, out_hbm)


_sc_call = functools.partial(
    pl.kernel,
    out_type=jax.ShapeDtypeStruct((16,), jnp.float32),
    mesh=plsc.VectorSubcoreMesh(core_axis_name="c", subcore_axis_name="s",
                                num_cores=_NC, num_subcores=_NS),
    scratch_types=[
        pltpu.VMEM((16,), jnp.float32),
        pltpu.VMEM((16,), jnp.int32),
        pltpu.VMEM((6, 256), jnp.float32),
        pltpu.VMEM((16,), jnp.int32),
        pltpu.VMEM((16,), jnp.float32),
        pltpu.VMEM((16,), jnp.float32),
        pltpu.SemaphoreType.DMA,
        pltpu.SemaphoreType.DMA,
    ],
)(_sc_kernel)


def kernel(s, g, s_idx, g_idx, ev_w1, ev_b1, ev_w2, ev_b2, ms_w1, ms_b1,
           ms_w2, ms_b2, states, value_function, num_cols):
    base_key = jax.random.key(42)

    mA, iA = _sample_middle_state(base_key, s, g, 0, ms_w1, ms_b1, ms_w2,
                                  ms_b2, states, num_cols)
    mB, iB = _sample_middle_state(base_key, s, mA, 1, ms_w1, ms_b1, ms_w2,
                                  ms_b2, states, num_cols)
    mC, iC = _sample_middle_state(base_key, mA, g, 2, ms_w1, ms_b1, ms_w2,
                                  ms_b2, states, num_cols)

    def _bf(v):
        return v.astype(jnp.bfloat16).astype(jnp.float32)

    sf = jnp.concatenate([
        _bf(s.reshape(-1)), _bf(g.reshape(-1)), _bf(mA.reshape(-1)),
        _bf(mB.reshape(-1)), _bf(mC.reshape(-1)), ev_b2.reshape(-1),
        jnp.zeros((5,), jnp.float32),
    ]).astype(jnp.float32)
    si = jnp.concatenate([
        s_idx.reshape(-1), g_idx.reshape(-1), iA.reshape(-1), iB.reshape(-1),
        iC.reshape(-1), jnp.zeros((11,), jnp.int32),
    ]).astype(jnp.int32)

    wpack = jnp.concatenate([
        _bf(ev_w1), ev_b1.reshape(1, -1), _bf(ev_w2.reshape(1, -1)),
    ], axis=0)
    out = _sc_call(sf, si, wpack, value_function.reshape(-1))

    y = out[0:1].reshape(1, 1)
    v = out[1:2]
    loss = out[2]
    return y, v, loss
